# baseline (device time: 35041 ns/iter reference)
import jax
import jax.numpy as jnp
from jax import lax
from jax.experimental import pallas as pl
from jax.experimental.pallas import tpu as pltpu

N_DEV = 4
HALO = 128
SCALE = 0.08838834764831843
NEG = -1e9
BF = jnp.bfloat16


def kernel(x, Wq, K_ext, V_ext, Wo):
    B, sq, d = x.shape
    _, skv, hq, dh = K_ext.shape
    ext = skv + 2 * HALO

    x2 = x.reshape(sq, d)
    k2 = K_ext.reshape(skv, hq * dh)
    v2 = V_ext.reshape(skv, hq * dh)

    def body(x_ref, wq_ref, k_ref, v_ref, wo_ref, out_ref,
             kfull, vfull, sstage, q_scr, ctx_scr, send_sems, recv_sems):
        me = lax.axis_index("i")
        left = me - 1
        right = me + 1
        not_first = me > 0
        not_last = me < N_DEV - 1

        sstage[0, :, :] = k_ref[0:HALO, :].astype(BF)
        sstage[1, :, :] = v_ref[0:HALO, :].astype(BF)
        sstage[2, :, :] = k_ref[skv - HALO:skv, :].astype(BF)
        sstage[3, :, :] = v_ref[skv - HALO:skv, :].astype(BF)

        def halo_rdma(slot, dst_full, dst_row0, ssem, rsem, target):
            return pltpu.make_async_remote_copy(
                src_ref=sstage.at[slot],
                dst_ref=dst_full.at[pl.ds(dst_row0, HALO)],
                send_sem=ssem,
                recv_sem=rsem,
                device_id=(target,),
                device_id_type=pl.DeviceIdType.MESH,
            )

        @pl.when(jnp.logical_not(not_first))
        def _():
            kfull[pl.ds(0, HALO)] = jnp.zeros((HALO, d), BF)
            vfull[pl.ds(0, HALO)] = jnp.zeros((HALO, d), BF)

        @pl.when(jnp.logical_not(not_last))
        def _():
            kfull[pl.ds(skv + HALO, HALO)] = jnp.zeros((HALO, d), BF)
            vfull[pl.ds(skv + HALO, HALO)] = jnp.zeros((HALO, d), BF)

        barrier_sem = pltpu.get_barrier_semaphore()

        @pl.when(not_first)
        def _():
            pl.semaphore_signal(barrier_sem, inc=1, device_id=(left,),
                                device_id_type=pl.DeviceIdType.MESH)

        @pl.when(not_last)
        def _():
            pl.semaphore_signal(barrier_sem, inc=1, device_id=(right,),
                                device_id_type=pl.DeviceIdType.MESH)

        @pl.when(not_first & not_last)
        def _():
            pl.semaphore_wait(barrier_sem, 2)

        @pl.when(jnp.logical_not(not_first & not_last))
        def _():
            pl.semaphore_wait(barrier_sem, 1)

        @pl.when(not_first)
        def _():
            halo_rdma(0, kfull, skv + HALO,
                      send_sems.at[0], recv_sems.at[1], left).start()
            halo_rdma(1, vfull, skv + HALO,
                      send_sems.at[1], recv_sems.at[3], left).start()

        @pl.when(not_last)
        def _():
            halo_rdma(2, kfull, 0,
                      send_sems.at[2], recv_sems.at[0], right).start()
            halo_rdma(3, vfull, 0,
                      send_sems.at[3], recv_sems.at[2], right).start()

        kfull[pl.ds(HALO, skv)] = k_ref[:, :].astype(BF)
        vfull[pl.ds(HALO, skv)] = v_ref[:, :].astype(BF)
        q_scr[:, :] = (jnp.dot(x_ref[:, :].astype(BF), wq_ref[:, :].astype(BF),
                               preferred_element_type=jnp.float32)
                       * SCALE).astype(BF)

        def attn_block(h, i0, nq, nk, mask):
            c0 = h * dh
            qh = q_scr[pl.ds(i0, nq), c0:c0 + dh]
            kh = kfull[pl.ds(i0, nk), c0:c0 + dh]
            vh = vfull[pl.ds(i0, nk), c0:c0 + dh]
            s = lax.dot_general(qh, kh, (((1,), (1,)), ((), ())),
                                preferred_element_type=jnp.float32)
            s = jnp.where(mask, s, NEG)
            m = jnp.max(s, axis=1, keepdims=True)
            w = jnp.exp(s - m)
            p = (w / jnp.sum(w, axis=1, keepdims=True)).astype(BF)
            ctx_scr[pl.ds(i0, nq), c0:c0 + dh] = lax.dot_general(
                p, vh, (((1,), (0,)), ((), ())),
                preferred_element_type=jnp.float32).astype(BF)

        MB = 2 * HALO
        EB = HALO
        def band(nq, nk):
            ii = lax.broadcasted_iota(jnp.int32, (nq, nk), 0)
            jj = lax.broadcasted_iota(jnp.int32, (nq, nk), 1)
            return (jj >= ii) & (jj <= ii + 2 * HALO), jj

        mask_mid, _ = band(MB, MB + 2 * HALO)
        band_edge, jj_e = band(EB, EB + 2 * HALO)
        mask_lo = band_edge & (jj_e >= jnp.where(me == 0, HALO, 0))
        mask_hi = band_edge & (
            jj_e < jnp.where(me == N_DEV - 1, 2 * HALO, EB + 2 * HALO))

        for h in range(hq):
            for m in range(3):
                attn_block(h, HALO + m * MB, MB, MB + 2 * HALO, mask_mid)

        @pl.when(not_first)
        def _():
            halo_rdma(0, kfull, 0,
                      send_sems.at[0], recv_sems.at[0], left).wait_recv()
            halo_rdma(1, vfull, 0,
                      send_sems.at[1], recv_sems.at[2], left).wait_recv()

        for h in range(hq):
            attn_block(h, 0, EB, EB + 2 * HALO, mask_lo)

        @pl.when(not_last)
        def _():
            halo_rdma(2, kfull, skv + HALO,
                      send_sems.at[2], recv_sems.at[1], right).wait_recv()
            halo_rdma(3, vfull, skv + HALO,
                      send_sems.at[3], recv_sems.at[3], right).wait_recv()

        for h in range(hq):
            attn_block(h, skv - EB, EB, EB + 2 * HALO, mask_hi)

        out_ref[:, :] = jnp.dot(ctx_scr[:, :], wo_ref[:, :].astype(BF),
                                preferred_element_type=jnp.float32)

        @pl.when(not_first)
        def _():
            halo_rdma(0, kfull, skv + HALO,
                      send_sems.at[0], recv_sems.at[1], left).wait_send()
            halo_rdma(1, vfull, skv + HALO,
                      send_sems.at[1], recv_sems.at[3], left).wait_send()

        @pl.when(not_last)
        def _():
            halo_rdma(2, kfull, 0,
                      send_sems.at[2], recv_sems.at[0], right).wait_send()
            halo_rdma(3, vfull, 0,
                      send_sems.at[3], recv_sems.at[2], right).wait_send()

    out = pl.pallas_call(
        body,
        out_shape=jax.ShapeDtypeStruct((sq, d), jnp.float32),
        in_specs=[pl.BlockSpec(memory_space=pltpu.VMEM)] * 5,
        out_specs=pl.BlockSpec(memory_space=pltpu.VMEM),
        scratch_shapes=[
            pltpu.VMEM((ext, d), BF),
            pltpu.VMEM((ext, d), BF),
            pltpu.VMEM((4, HALO, d), BF),
            pltpu.VMEM((sq, d), BF),
            pltpu.VMEM((sq, d), BF),
            pltpu.SemaphoreType.DMA((4,)),
            pltpu.SemaphoreType.DMA((4,)),
        ],
        compiler_params=pltpu.CompilerParams(collective_id=0),
    )(x2, Wq, k2, v2, Wo)
    return out.reshape(B, sq, d)


# device time: 29468 ns/iter; 1.1891x vs baseline; 1.1891x over previous
import jax
import jax.numpy as jnp
from jax import lax
from jax.experimental import pallas as pl
from jax.experimental.pallas import tpu as pltpu

N_DEV = 4
HALO = 128
SCALE = 0.08838834764831843
LOG2E = 1.4426950408889634
NEG = -1e9
BF = jnp.bfloat16


def kernel(x, Wq, K_ext, V_ext, Wo):
    B, sq, d = x.shape
    _, skv, hq, dh = K_ext.shape
    ext = skv + 2 * HALO

    xb = x.reshape(sq, d).astype(BF)
    wqb = Wq.astype(BF)
    wob = Wo.astype(BF)
    kb = K_ext.reshape(skv, hq * dh).astype(BF)
    vb = V_ext.reshape(skv, hq * dh).astype(BF)

    def body(x_ref, wq_ref, k_ref, v_ref, wo_ref, out_ref,
             kfull, vfull, q_scr, ctx_scr, send_sems, recv_sems):
        me = lax.axis_index("i")
        left = me - 1
        right = me + 1
        not_first = me > 0
        not_last = me < N_DEV - 1

        def halo_rdma(src_ref, row0, dst_full, dst_row0, ssem, rsem, target):
            return pltpu.make_async_remote_copy(
                src_ref=src_ref.at[pl.ds(row0, HALO)],
                dst_ref=dst_full.at[pl.ds(dst_row0, HALO)],
                send_sem=ssem,
                recv_sem=rsem,
                device_id=(target,),
                device_id_type=pl.DeviceIdType.MESH,
            )

        @pl.when(jnp.logical_not(not_first))
        def _():
            kfull[pl.ds(0, HALO)] = jnp.zeros((HALO, d), BF)
            vfull[pl.ds(0, HALO)] = jnp.zeros((HALO, d), BF)

        @pl.when(jnp.logical_not(not_last))
        def _():
            kfull[pl.ds(skv + HALO, HALO)] = jnp.zeros((HALO, d), BF)
            vfull[pl.ds(skv + HALO, HALO)] = jnp.zeros((HALO, d), BF)

        barrier_sem = pltpu.get_barrier_semaphore()

        @pl.when(not_first)
        def _():
            pl.semaphore_signal(barrier_sem, inc=1, device_id=(left,),
                                device_id_type=pl.DeviceIdType.MESH)

        @pl.when(not_last)
        def _():
            pl.semaphore_signal(barrier_sem, inc=1, device_id=(right,),
                                device_id_type=pl.DeviceIdType.MESH)

        @pl.when(not_first & not_last)
        def _():
            pl.semaphore_wait(barrier_sem, 2)

        @pl.when(jnp.logical_not(not_first & not_last))
        def _():
            pl.semaphore_wait(barrier_sem, 1)

        @pl.when(not_first)
        def _():
            halo_rdma(k_ref, 0, kfull, skv + HALO,
                      send_sems.at[0], recv_sems.at[1], left).start()
            halo_rdma(v_ref, 0, vfull, skv + HALO,
                      send_sems.at[1], recv_sems.at[3], left).start()

        @pl.when(not_last)
        def _():
            halo_rdma(k_ref, skv - HALO, kfull, 0,
                      send_sems.at[2], recv_sems.at[0], right).start()
            halo_rdma(v_ref, skv - HALO, vfull, 0,
                      send_sems.at[3], recv_sems.at[2], right).start()

        kfull[pl.ds(HALO, skv)] = k_ref[:, :]
        vfull[pl.ds(HALO, skv)] = v_ref[:, :]
        q_scr[:, :] = (jnp.dot(x_ref[:, :], wq_ref[:, :],
                               preferred_element_type=jnp.float32)
                       * (SCALE * LOG2E)).astype(BF)

        def attn_block(h, i0, nq, nk, bias):
            c0 = h * dh
            qh = q_scr[pl.ds(i0, nq), c0:c0 + dh]
            kh = kfull[pl.ds(i0, nk), c0:c0 + dh]
            vh = vfull[pl.ds(i0, nk), c0:c0 + dh]
            s = lax.dot_general(qh, kh, (((1,), (1,)), ((), ())),
                                preferred_element_type=jnp.float32)
            w = jnp.exp2(s + bias)
            denom = jnp.sum(w, axis=1, keepdims=True)
            ctx = lax.dot_general(w.astype(BF), vh, (((1,), (0,)), ((), ())),
                                  preferred_element_type=jnp.float32)
            ctx_scr[pl.ds(i0, nq), c0:c0 + dh] = \
                (ctx * (1.0 / denom)).astype(BF)

        MB = 2 * HALO
        EB = HALO
        def band(nq, nk):
            ii = lax.broadcasted_iota(jnp.int32, (nq, nk), 0)
            jj = lax.broadcasted_iota(jnp.int32, (nq, nk), 1)
            return (jj >= ii) & (jj <= ii + 2 * HALO), jj

        band_mid, _ = band(MB, MB + 2 * HALO)
        band_edge, jj_e = band(EB, EB + 2 * HALO)
        zero = jnp.float32(0.0)
        bias_mid = jnp.where(band_mid, zero, NEG)
        bias_lo = jnp.where(
            band_edge & (jj_e >= jnp.where(me == 0, HALO, 0)), zero, NEG)
        bias_hi = jnp.where(
            band_edge & (jj_e < jnp.where(me == N_DEV - 1,
                                          2 * HALO, EB + 2 * HALO)),
            zero, NEG)

        for h in range(hq):
            for m in range(3):
                attn_block(h, HALO + m * MB, MB, MB + 2 * HALO, bias_mid)

        @pl.when(not_first)
        def _():
            halo_rdma(k_ref, 0, kfull, 0,
                      send_sems.at[0], recv_sems.at[0], left).wait_recv()
            halo_rdma(v_ref, 0, vfull, 0,
                      send_sems.at[1], recv_sems.at[2], left).wait_recv()

        for h in range(hq):
            attn_block(h, 0, EB, EB + 2 * HALO, bias_lo)

        @pl.when(not_last)
        def _():
            halo_rdma(k_ref, 0, kfull, skv + HALO,
                      send_sems.at[2], recv_sems.at[1], right).wait_recv()
            halo_rdma(v_ref, 0, vfull, skv + HALO,
                      send_sems.at[3], recv_sems.at[3], right).wait_recv()

        for h in range(hq):
            attn_block(h, skv - EB, EB, EB + 2 * HALO, bias_hi)

        out_ref[:, :] = jnp.dot(ctx_scr[:, :], wo_ref[:, :],
                                preferred_element_type=jnp.float32)

        @pl.when(not_first)
        def _():
            halo_rdma(k_ref, 0, kfull, skv + HALO,
                      send_sems.at[0], recv_sems.at[1], left).wait_send()
            halo_rdma(v_ref, 0, vfull, skv + HALO,
                      send_sems.at[1], recv_sems.at[3], left).wait_send()

        @pl.when(not_last)
        def _():
            halo_rdma(k_ref, skv - HALO, kfull, 0,
                      send_sems.at[2], recv_sems.at[0], right).wait_send()
            halo_rdma(v_ref, skv - HALO, vfull, 0,
                      send_sems.at[3], recv_sems.at[2], right).wait_send()

    out = pl.pallas_call(
        body,
        out_shape=jax.ShapeDtypeStruct((sq, d), jnp.float32),
        in_specs=[pl.BlockSpec(memory_space=pltpu.VMEM)] * 5,
        out_specs=pl.BlockSpec(memory_space=pltpu.VMEM),
        scratch_shapes=[
            pltpu.VMEM((ext, d), BF),
            pltpu.VMEM((ext, d), BF),
            pltpu.VMEM((sq, d), BF),
            pltpu.VMEM((sq, d), BF),
            pltpu.SemaphoreType.DMA((4,)),
            pltpu.SemaphoreType.DMA((4,)),
        ],
        compiler_params=pltpu.CompilerParams(collective_id=0),
    )(xb, wqb, kb, vb, wob)
    return out.reshape(B, sq, d)


# device time: 27016 ns/iter; 1.2970x vs baseline; 1.0908x over previous
import jax
import jax.numpy as jnp
from jax import lax
from jax.experimental import pallas as pl
from jax.experimental.pallas import tpu as pltpu

N_DEV = 4
HALO = 128
SCALE = 0.08838834764831843
LOG2E = 1.4426950408889634
NEG = -1e9
BF = jnp.bfloat16


def kernel(x, Wq, K_ext, V_ext, Wo):
    B, sq, d = x.shape
    _, skv, hq, dh = K_ext.shape
    ext = skv + 2 * HALO

    k2 = K_ext.reshape(skv, hq * dh)
    v2 = V_ext.reshape(skv, hq * dh)

    def body(x_ref, wq_ref, k_ref, v_ref, wo_ref, out_ref,
             kfull, vfull, sstage, q_scr, ctx_scr, send_sems, recv_sems):
        me = lax.axis_index("i")
        left = me - 1
        right = me + 1
        not_first = me > 0
        not_last = me < N_DEV - 1

        sstage[0, :, :] = k_ref[0:HALO, :].astype(BF)
        sstage[1, :, :] = v_ref[0:HALO, :].astype(BF)
        sstage[2, :, :] = k_ref[skv - HALO:skv, :].astype(BF)
        sstage[3, :, :] = v_ref[skv - HALO:skv, :].astype(BF)

        def halo_rdma(slot, dst_full, dst_row0, ssem, rsem, target):
            return pltpu.make_async_remote_copy(
                src_ref=sstage.at[slot],
                dst_ref=dst_full.at[pl.ds(dst_row0, HALO)],
                send_sem=ssem,
                recv_sem=rsem,
                device_id=(target,),
                device_id_type=pl.DeviceIdType.MESH,
            )

        @pl.when(jnp.logical_not(not_first))
        def _():
            kfull[pl.ds(0, HALO)] = jnp.zeros((HALO, d), BF)
            vfull[pl.ds(0, HALO)] = jnp.zeros((HALO, d), BF)

        @pl.when(jnp.logical_not(not_last))
        def _():
            kfull[pl.ds(skv + HALO, HALO)] = jnp.zeros((HALO, d), BF)
            vfull[pl.ds(skv + HALO, HALO)] = jnp.zeros((HALO, d), BF)

        barrier_sem = pltpu.get_barrier_semaphore()

        @pl.when(not_first)
        def _():
            pl.semaphore_signal(barrier_sem, inc=1, device_id=(left,),
                                device_id_type=pl.DeviceIdType.MESH)

        @pl.when(not_last)
        def _():
            pl.semaphore_signal(barrier_sem, inc=1, device_id=(right,),
                                device_id_type=pl.DeviceIdType.MESH)

        @pl.when(not_first & not_last)
        def _():
            pl.semaphore_wait(barrier_sem, 2)

        @pl.when(jnp.logical_not(not_first & not_last))
        def _():
            pl.semaphore_wait(barrier_sem, 1)

        @pl.when(not_first)
        def _():
            halo_rdma(0, kfull, skv + HALO,
                      send_sems.at[0], recv_sems.at[1], left).start()
            halo_rdma(1, vfull, skv + HALO,
                      send_sems.at[1], recv_sems.at[3], left).start()

        @pl.when(not_last)
        def _():
            halo_rdma(2, kfull, 0,
                      send_sems.at[2], recv_sems.at[0], right).start()
            halo_rdma(3, vfull, 0,
                      send_sems.at[3], recv_sems.at[2], right).start()

        kfull[pl.ds(HALO, skv)] = k_ref[:, :].astype(BF)
        vfull[pl.ds(HALO, skv)] = v_ref[:, :].astype(BF)
        q_scr[:, :] = (jnp.dot(x_ref[0].astype(BF), wq_ref[:, :].astype(BF),
                               preferred_element_type=jnp.float32)
                       * (SCALE * LOG2E)).astype(BF)

        def attn_block(h, i0, nq, nk, bias):
            c0 = h * dh
            qh = q_scr[pl.ds(i0, nq), c0:c0 + dh]
            kh = kfull[pl.ds(i0, nk), c0:c0 + dh]
            vh = vfull[pl.ds(i0, nk), c0:c0 + dh]
            s = lax.dot_general(qh, kh, (((1,), (1,)), ((), ())),
                                preferred_element_type=jnp.float32)
            w = jnp.exp2(s + bias)
            denom = jnp.sum(w, axis=1, keepdims=True)
            ctx = lax.dot_general(w.astype(BF), vh, (((1,), (0,)), ((), ())),
                                  preferred_element_type=jnp.float32)
            ctx_scr[pl.ds(i0, nq), c0:c0 + dh] = \
                (ctx * (1.0 / denom)).astype(BF)

        MB = 2 * HALO
        EB = HALO
        def band(nq, nk):
            ii = lax.broadcasted_iota(jnp.int32, (nq, nk), 0)
            jj = lax.broadcasted_iota(jnp.int32, (nq, nk), 1)
            return (jj >= ii) & (jj <= ii + 2 * HALO), jj

        band_mid, _ = band(MB, MB + 2 * HALO)
        band_edge, jj_e = band(EB, EB + 2 * HALO)
        zero = jnp.float32(0.0)
        bias_mid = jnp.where(band_mid, zero, NEG)
        bias_lo = jnp.where(
            band_edge & (jj_e >= jnp.where(me == 0, HALO, 0)), zero, NEG)
        bias_hi = jnp.where(
            band_edge & (jj_e < jnp.where(me == N_DEV - 1,
                                          2 * HALO, EB + 2 * HALO)),
            zero, NEG)

        for h in range(hq):
            for m in range(3):
                attn_block(h, HALO + m * MB, MB, MB + 2 * HALO, bias_mid)

        @pl.when(not_first)
        def _():
            halo_rdma(0, kfull, 0,
                      send_sems.at[0], recv_sems.at[0], left).wait_recv()
            halo_rdma(1, vfull, 0,
                      send_sems.at[1], recv_sems.at[2], left).wait_recv()

        for h in range(hq):
            attn_block(h, 0, EB, EB + 2 * HALO, bias_lo)

        @pl.when(not_last)
        def _():
            halo_rdma(2, kfull, skv + HALO,
                      send_sems.at[2], recv_sems.at[1], right).wait_recv()
            halo_rdma(3, vfull, skv + HALO,
                      send_sems.at[3], recv_sems.at[3], right).wait_recv()

        for h in range(hq):
            attn_block(h, skv - EB, EB, EB + 2 * HALO, bias_hi)

        out_ref[0] = jnp.dot(ctx_scr[:, :], wo_ref[:, :].astype(BF),
                             preferred_element_type=jnp.float32)

        @pl.when(not_first)
        def _():
            halo_rdma(0, kfull, skv + HALO,
                      send_sems.at[0], recv_sems.at[1], left).wait_send()
            halo_rdma(1, vfull, skv + HALO,
                      send_sems.at[1], recv_sems.at[3], left).wait_send()

        @pl.when(not_last)
        def _():
            halo_rdma(2, kfull, 0,
                      send_sems.at[2], recv_sems.at[0], right).wait_send()
            halo_rdma(3, vfull, 0,
                      send_sems.at[3], recv_sems.at[2], right).wait_send()

    out = pl.pallas_call(
        body,
        out_shape=jax.ShapeDtypeStruct((B, sq, d), jnp.float32),
        in_specs=[pl.BlockSpec(memory_space=pltpu.VMEM)] * 5,
        out_specs=pl.BlockSpec(memory_space=pltpu.VMEM),
        scratch_shapes=[
            pltpu.VMEM((ext, d), BF),
            pltpu.VMEM((ext, d), BF),
            pltpu.VMEM((4, HALO, d), BF),
            pltpu.VMEM((sq, d), BF),
            pltpu.VMEM((sq, d), BF),
            pltpu.SemaphoreType.DMA((4,)),
            pltpu.SemaphoreType.DMA((4,)),
        ],
        compiler_params=pltpu.CompilerParams(collective_id=0),
    )(x, Wq, k2, v2, Wo)
    return out
